# baseline (speedup 1.0000x reference)
import functools

import jax
import jax.numpy as jnp
from jax import lax
from jax.experimental import pallas as pl
from jax.experimental.pallas import tpu as pltpu
from jax.experimental.pallas import tpu_sc as plsc

_N = 4096
_C = 1024
_F = 4096
_MBM = 16
_BM = 128
_MB = _N // _BM
_R = _BM // _MBM
_NMB = _N // _MBM
_K = 1024
_TB = 256
_NT = _N // _TB

_INTERPRET = False


def _stage_a_kernel(x_ref, b1_ref, w1_hbm, bmc_hbm, mdiff_ref,
                    bm_ref, w1_ref, bmc_ref, sem_w1, sem_bmc):
    i = pl.program_id(0)

    @pl.when(i == 0)
    def _start_dma():
        pltpu.make_async_copy(w1_hbm, w1_ref, sem_w1).start()
        pltpu.make_async_copy(bmc_hbm, bmc_ref, sem_bmc).start()

    @pl.when(i < _NT)
    def _phase1():
        bm_ref[pl.ds(_MBM * i, _MBM), :] = (
            x_ref[...].reshape(_MBM, _MBM, _C).mean(axis=1))

    @pl.when(i == _NT)
    def _mdiff():
        pltpu.make_async_copy(w1_hbm, w1_ref, sem_w1).wait()
        pltpu.make_async_copy(bmc_hbm, bmc_ref, sem_bmc).wait()
        t = jax.lax.dot_general(bm_ref[...], w1_ref[...],
                                (((1,), (1,)), ((), ())),
                                preferred_element_type=jnp.float32)
        t = t + b1_ref[...]
        md = jnp.abs(t - bmc_ref[...])
        mdiff_ref[...] = md.reshape(_MB, _R, _F).sum(axis=1)


def _sc_thresholds(mdiff):
    mesh = plsc.VectorSubcoreMesh(core_axis_name="c", subcore_axis_name="s")

    @functools.partial(
        pl.kernel, mesh=mesh,
        out_type=jax.ShapeDtypeStruct((_MB, 16), jnp.int32),
        scratch_types=[
            pltpu.VMEM((_F,), jnp.int32),
            pltpu.VMEM((16,), jnp.int32),
            pltpu.VMEM((48,), jnp.float32),
        ],
    )
    def body(mdiff_hbm, thr_hbm, row_v, thr_v, tsum_v):
        wid = lax.axis_index("s") * 2 + lax.axis_index("c")
        pltpu.sync_copy(mdiff_hbm.at[wid], row_v)
        ones = jnp.ones((16,), jnp.float32)
        zeros = jnp.zeros((16,), jnp.float32)
        kf = jnp.full((16,), float(_K), jnp.float32)
        lanes = lax.iota(jnp.int32, 16)
        tsum_v[pl.ds(0, 16)] = zeros
        tsum_v[pl.ds(32, 16)] = zeros

        def it(_, carry):
            lo, hi = carry
            mid = lo + lax.shift_right_logical(hi - lo, 1)

            def chunk(c, acc):
                v = row_v[pl.ds(c * 16, 16)]
                return acc + jnp.where(v >= mid, ones, zeros)

            tot = lax.fori_loop(0, _F // 16, chunk, zeros)
            for sh in (8, 4, 2, 1):
                tsum_v[pl.ds(16, 16)] = tot
                wp = tsum_v[pl.ds(16 + sh, 16)]
                wm = tsum_v[pl.ds(16 - sh, 16)]
                take_m = jnp.bitwise_and(lanes, sh) != 0
                tot = tot + jnp.where(take_m, wm, wp)
            gev = tot >= kf
            return jnp.where(gev, mid, lo), jnp.where(gev, hi, mid)

        lo0 = jnp.zeros((16,), jnp.int32)
        hi0 = jnp.full((16,), 0x7F800000, jnp.int32)
        lo, _hi = lax.fori_loop(0, 31, it, (lo0, hi0))
        thr_v[...] = lo
        pltpu.sync_copy(thr_v, thr_hbm.at[wid])

    return body(mdiff)


def _stage_b_kernel(x_ref, b1_ref, pa_ref, mdiff_ref, thr_ref, w2_hbm,
                    w1_hbm, oc_ref, out_ref, w1_ref, w2_ref, sem_w1, sem_w2):
    m = pl.program_id(0)

    @pl.when(m == 0)
    def _start_dma():
        pltpu.make_async_copy(w1_hbm, w1_ref, sem_w1).start()
        pltpu.make_async_copy(w2_hbm, w2_ref, sem_w2).start()
        pltpu.make_async_copy(w1_hbm, w1_ref, sem_w1).wait()

    mid = jax.lax.dot_general(x_ref[...], w1_ref[...],
                              (((1,), (1,)), ((), ())),
                              preferred_element_type=jnp.float32)
    mid = mid + b1_ref[...]
    act = jax.nn.gelu(mid)

    @pl.when(m == 0)
    def _wait_w2():
        pltpu.make_async_copy(w2_hbm, w2_ref, sem_w2).wait()
    bits0 = jax.lax.bitcast_convert_type(mdiff_ref[pl.ds(2 * m, 1), :],
                                           jnp.int32)
    bits1 = jax.lax.bitcast_convert_type(mdiff_ref[pl.ds(2 * m + 1, 1), :],
                                         jnp.int32)
    m0 = (bits0 >= thr_ref[pl.ds(2 * m, 1), pl.ds(0, 1)]).astype(jnp.float32)
    m1 = (bits1
          >= thr_ref[pl.ds(2 * m + 1, 1), pl.ds(0, 1)]).astype(jnp.float32)
    condf = (jax.lax.broadcasted_iota(jnp.int32, (_TB, 1), 0)
             < _BM).astype(jnp.float32)
    mask = m0 * condf + m1 * (1.0 - condf)
    delta = (act - pa_ref[...]) * mask
    part = jax.lax.dot_general(delta, w2_ref[...],
                               (((1,), (1,)), ((), ())),
                               preferred_element_type=jnp.float32)
    out_ref[...] = oc_ref[...] + part


def kernel(x, W1, b1, W2, b2, blockmean_mid_cache, pa_cache, out_cache):
    x2 = x.reshape(_N, _C)
    bmc = blockmean_mid_cache.reshape(_NMB, _F)
    b1r = b1.reshape(1, _F)
    pa2 = pa_cache.reshape(_N, _F)
    oc2 = out_cache.reshape(_N, _C)

    mdiff = pl.pallas_call(
        _stage_a_kernel,
        grid=(_NT + 1,),
        in_specs=[
            pl.BlockSpec((_TB, _C), lambda i: (jnp.minimum(i, _NT - 1), 0)),
            pl.BlockSpec((1, _F), lambda i: (0, 0)),
            pl.BlockSpec(memory_space=pl.ANY),
            pl.BlockSpec(memory_space=pl.ANY),
        ],
        out_specs=pl.BlockSpec((_MB, _F), lambda i: (0, 0)),
        out_shape=jax.ShapeDtypeStruct((_MB, _F), jnp.float32),
        scratch_shapes=[
            pltpu.VMEM((_NMB, _C), jnp.float32),
            pltpu.VMEM((_F, _C), jnp.float32),
            pltpu.VMEM((_NMB, _F), jnp.float32),
            pltpu.SemaphoreType.DMA,
            pltpu.SemaphoreType.DMA,
        ],
        compiler_params=pltpu.CompilerParams(
            dimension_semantics=("arbitrary",)),
        interpret=_INTERPRET,
    )(x2, b1r, W1, bmc)

    thr = _sc_thresholds(jax.lax.bitcast_convert_type(mdiff, jnp.int32))

    out = pl.pallas_call(
        _stage_b_kernel,
        grid=(_NT,),
        in_specs=[
            pl.BlockSpec((_TB, _C), lambda m: (m, 0)),
            pl.BlockSpec((1, _F), lambda m: (0, 0)),
            pl.BlockSpec((_TB, _F), lambda m: (m, 0)),
            pl.BlockSpec((_MB, _F), lambda m: (0, 0)),
            pl.BlockSpec((_MB, 16), lambda m: (0, 0)),
            pl.BlockSpec(memory_space=pl.ANY),
            pl.BlockSpec(memory_space=pl.ANY),
            pl.BlockSpec((_TB, _C), lambda m: (m, 0)),
        ],
        out_specs=pl.BlockSpec((_TB, _C), lambda m: (m, 0)),
        out_shape=jax.ShapeDtypeStruct((_N, _C), jnp.float32),
        scratch_shapes=[
            pltpu.VMEM((_F, _C), jnp.float32),
            pltpu.VMEM((_C, _F), jnp.float32),
            pltpu.SemaphoreType.DMA,
            pltpu.SemaphoreType.DMA,
        ],
        compiler_params=pltpu.CompilerParams(
            dimension_semantics=("arbitrary",),
            vmem_limit_bytes=100 * 1024 * 1024),
        interpret=_INTERPRET,
    )(x2, b1r, pa2, mdiff, thr, W2, W1, oc2)

    return out.reshape(1, _N, _C)


# speedup vs baseline: 1.5406x; 1.5406x over previous
import jax
import jax.numpy as jnp
from jax.experimental import pallas as pl
from jax.experimental.pallas import tpu as pltpu

_N = 4096
_C = 1024
_F = 4096
_MBM = 16
_BM = 128
_MB = _N // _BM
_R = _BM // _MBM
_NMB = _N // _MBM
_K = 1024
_TB = 256
_NT = _N // _TB

def _fused_kernel(x_ref, b1_ref, w1_hbm, bmc_hbm, pa_ref, w2_hbm, oc_ref,
                  out_ref, bm_ref, mdiff_ref, thr_ref, xs_ref,
                  w1_ref, w2_ref, bmc_ref, sem_w1, sem_w2, sem_bmc):
    i = pl.program_id(0)

    @pl.when(i == 0)
    def _start_weight_dma():
        pltpu.make_async_copy(w1_hbm, w1_ref, sem_w1).start()
        pltpu.make_async_copy(w2_hbm, w2_ref, sem_w2).start()
        pltpu.make_async_copy(bmc_hbm, bmc_ref, sem_bmc).start()

    @pl.when(i < _NT)
    def _phase1():
        xv = x_ref[...]
        xs_ref[pl.ds(_TB * i, _TB), :] = xv.astype(jnp.bfloat16)
        bm_ref[pl.ds(_MBM * i, _MBM), :] = (
            xv.reshape(_MBM, _MBM, _C).mean(axis=1))

    @pl.when(i == _NT)
    def _select():
        pltpu.make_async_copy(w1_hbm, w1_ref, sem_w1).wait()
        pltpu.make_async_copy(bmc_hbm, bmc_ref, sem_bmc).wait()
        t = jax.lax.dot_general(bm_ref[...], w1_ref[...],
                                (((1,), (1,)), ((), ())),
                                preferred_element_type=jnp.float32)
        t = t + b1_ref[...]
        md = jnp.abs(t - bmc_ref[...])
        mdiff_ref[...] = md.reshape(_MB, _R, _F).sum(axis=1)
        bits = jax.lax.bitcast_convert_type(mdiff_ref[...], jnp.int32)

        def body(_, carry):
            lo, hi = carry
            mid = lo + (hi - lo) // 2
            cnt = jnp.sum((bits >= mid).astype(jnp.int32), axis=1,
                          keepdims=True)
            ge = cnt >= _K
            return jnp.where(ge, mid, lo), jnp.where(ge, hi, mid)

        lo0 = jnp.zeros((_MB, 1), jnp.int32)
        hi0 = jnp.full((_MB, 1), 0x7F800000, jnp.int32)
        lo, _hi = jax.lax.fori_loop(0, 31, body, (lo0, hi0))
        thr_ref[...] = jax.lax.bitcast_convert_type(lo, jnp.float32)
        pltpu.make_async_copy(w2_hbm, w2_ref, sem_w2).wait()

    @pl.when(i > _NT)
    def _phase2():
        m = i - (_NT + 1)
        xv = xs_ref[pl.ds(_TB * m, _TB), :].astype(jnp.float32)
        mid = jax.lax.dot_general(xv, w1_ref[...],
                                  (((1,), (1,)), ((), ())),
                                  preferred_element_type=jnp.float32)
        mid = mid + b1_ref[...]
        act = jax.nn.gelu(mid)
        m0 = (mdiff_ref[pl.ds(2 * m, 1), :]
              >= thr_ref[pl.ds(2 * m, 1), :]).astype(jnp.float32)
        m1 = (mdiff_ref[pl.ds(2 * m + 1, 1), :]
              >= thr_ref[pl.ds(2 * m + 1, 1), :]).astype(jnp.float32)
        condf = (jax.lax.broadcasted_iota(jnp.int32, (_TB, 1), 0)
                 < _BM).astype(jnp.float32)
        mask = m0 * condf + m1 * (1.0 - condf)
        delta = (act - pa_ref[...]) * mask
        part = jax.lax.dot_general(delta, w2_ref[...],
                                   (((1,), (1,)), ((), ())),
                                   preferred_element_type=jnp.float32)
        out_ref[...] = oc_ref[...] + part


def kernel(x, W1, b1, W2, b2, blockmean_mid_cache, pa_cache, out_cache):
    x2 = x.reshape(_N, _C)
    bmc = blockmean_mid_cache.reshape(_NMB, _F)
    b1r = b1.reshape(1, _F)
    pa2 = pa_cache.reshape(_N, _F)
    oc2 = out_cache.reshape(_N, _C)

    def _xmap(i):
        return (jnp.minimum(i, _NT - 1), 0)

    def _p2map(i):
        return (jnp.maximum(i - (_NT + 1), 0), 0)

    out = pl.pallas_call(
        _fused_kernel,
        grid=(2 * _NT + 1,),
        in_specs=[
            pl.BlockSpec((_TB, _C), _xmap),
            pl.BlockSpec((1, _F), lambda i: (0, 0)),
            pl.BlockSpec(memory_space=pl.ANY),
            pl.BlockSpec(memory_space=pl.ANY),
            pl.BlockSpec((_TB, _F), _p2map),
            pl.BlockSpec(memory_space=pl.ANY),
            pl.BlockSpec((_TB, _C), _p2map),
        ],
        out_specs=pl.BlockSpec((_TB, _C), _p2map),
        out_shape=jax.ShapeDtypeStruct((_N, _C), jnp.float32),
        scratch_shapes=[
            pltpu.VMEM((_NMB, _C), jnp.float32),
            pltpu.VMEM((_MB, _F), jnp.float32),
            pltpu.VMEM((_MB, 1), jnp.float32),
            pltpu.VMEM((_N, _C), jnp.bfloat16),
            pltpu.VMEM((_F, _C), jnp.float32),
            pltpu.VMEM((_C, _F), jnp.float32),
            pltpu.VMEM((_NMB, _F), jnp.float32),
            pltpu.SemaphoreType.DMA,
            pltpu.SemaphoreType.DMA,
            pltpu.SemaphoreType.DMA,
        ],
        compiler_params=pltpu.CompilerParams(
            dimension_semantics=("arbitrary",),
            vmem_limit_bytes=100 * 1024 * 1024),
    )(x2, b1r, W1, bmc, pa2, W2, oc2)

    return out.reshape(1, _N, _C)
